# baseline (device time: 56841 ns/iter reference)
import jax
import jax.numpy as jnp
from jax import lax
from jax.experimental import pallas as pl
from jax.experimental.pallas import tpu as pltpu

N_Y = 4
V_PER = 8192
HALF = 512
C = 128
SC = 64
K = 32


def kernel(ids, E):
    my_y = lax.axis_index("y")
    my_x = lax.axis_index("x")

    my_ids = lax.dynamic_slice(ids, (my_x * HALF,), (HALF,))
    local = my_ids - my_y * V_PER
    in_range = (local >= 0) & (local < V_PER)
    safe = jnp.clip(local, 0, V_PER - 1).astype(jnp.int32)
    scale = in_range.astype(jnp.float32)[:, None]

    t, d = ids.shape[0], E.shape[1]

    def body(ids_ref, scale_ref, e_ref, out_ref,
             fbuf, acc_ref, gbuf, xbuf, rbuf, gsem,
             rs_send, rs_recv, ag_send, ag_recv, x_send, x_recv):
        xx = lax.axis_index("x")
        yy = lax.axis_index("y")
        zz = lax.axis_index("z")
        right = (yy + 1) % N_Y
        left = (yy - 1) % N_Y
        mine0 = xx * HALF
        theirs0 = (1 - xx) * HALF

        barrier_sem = pltpu.get_barrier_semaphore()
        for nbr in ((xx, left, zz), (xx, right, zz), (1 - xx, yy, zz)):
            pl.semaphore_signal(
                barrier_sem, inc=1,
                device_id=nbr, device_id_type=pl.DeviceIdType.MESH,
            )
        pl.semaphore_wait(barrier_sem, 3)

        def row_dma(row):
            return pltpu.make_async_copy(
                e_ref.at[pl.ds(ids_ref[row], 1), :],
                fbuf.at[pl.ds(row, 1), :],
                gsem.at[row % K],
            )

        def gather_chunk(j):
            base = ((yy - j) % N_Y) * C

            def gbody(i, _):
                row_dma(base + i).start()

                @pl.when(i >= K)
                def _():
                    row_dma(base + i - K).wait()

                return 0

            lax.fori_loop(0, C, gbody, 0, unroll=8)
            for jj in range(K):
                row_dma(base + C - K + jj).wait()
            acc_ref[pl.ds(base, C), :] = (
                fbuf[pl.ds(base, C), :] * scale_ref[pl.ds(base, C), :]
            ).astype(jnp.bfloat16)

        pending = []
        x_slots = []

        def x_forward(chunk, slot):
            for h in range(2):
                rd = pltpu.make_async_remote_copy(
                    src_ref=gbuf.at[pl.ds(chunk * C + h * SC, SC), :],
                    dst_ref=xbuf.at[pl.ds(chunk * C + h * SC, SC), :],
                    send_sem=x_send.at[2 * slot + h],
                    recv_sem=x_recv.at[2 * slot + h],
                    device_id=(1 - xx, yy, zz),
                    device_id_type=pl.DeviceIdType.MESH,
                )
                rd.start()
                x_slots.append((rd, chunk * C + h * SC))

        gather_chunk(0)
        for p in range(N_Y - 1):
            s = (yy - p) % N_Y
            r = (yy - p - 1) % N_Y
            rd = pltpu.make_async_remote_copy(
                src_ref=acc_ref.at[pl.ds(s * C, C), :],
                dst_ref=rbuf.at[p],
                send_sem=rs_send.at[p],
                recv_sem=rs_recv.at[p],
                device_id=(xx, right, zz),
                device_id_type=pl.DeviceIdType.MESH,
            )
            rd.start()
            gather_chunk(p + 1)
            rd.wait()
            acc_ref[pl.ds(r * C, C), :] = (
                acc_ref[pl.ds(r * C, C), :] + rbuf[p]
            )

        own = (yy + 1) % N_Y
        gbuf[pl.ds(own * C, C), :] = acc_ref[pl.ds(own * C, C), :]
        x_forward(own, 0)
        out_ref[pl.ds(mine0 + own * C, C), :] = (
            gbuf[pl.ds(own * C, C), :].astype(jnp.float32)
        )

        for p in range(N_Y - 1):
            a = (own - p) % N_Y
            g = (yy - p) % N_Y
            rds = []
            for h in range(2):
                rd = pltpu.make_async_remote_copy(
                    src_ref=gbuf.at[pl.ds(a * C + h * SC, SC), :],
                    dst_ref=gbuf.at[pl.ds(a * C + h * SC, SC), :],
                    send_sem=ag_send.at[2 * p + h],
                    recv_sem=ag_recv.at[2 * p + h],
                    device_id=(xx, right, zz),
                    device_id_type=pl.DeviceIdType.MESH,
                )
                rd.start()
                rds.append(rd)
                pending.append(rd)
            for h in range(2):
                rds[h].wait_recv()
                row = g * C + h * SC
                out_ref[pl.ds(mine0 + row, SC), :] = (
                    gbuf[pl.ds(row, SC), :].astype(jnp.float32)
                )
            x_forward(g, p + 1)

        for rd, row in x_slots:
            rd.wait_recv()
            out_ref[pl.ds(theirs0 + row, SC), :] = (
                xbuf[pl.ds(row, SC), :].astype(jnp.float32)
            )
        for rd, _ in x_slots:
            rd.wait_send()
        for rd in pending:
            rd.wait_send()

    return pl.pallas_call(
        body,
        out_shape=jax.ShapeDtypeStruct((t, d), jnp.float32),
        in_specs=[
            pl.BlockSpec(memory_space=pltpu.SMEM),
            pl.BlockSpec(memory_space=pltpu.VMEM),
            pl.BlockSpec(memory_space=pl.ANY),
        ],
        out_specs=pl.BlockSpec(memory_space=pltpu.VMEM),
        scratch_shapes=[
            pltpu.VMEM((HALF, d), jnp.float32),
            pltpu.VMEM((HALF, d), jnp.bfloat16),
            pltpu.VMEM((HALF, d), jnp.bfloat16),
            pltpu.VMEM((HALF, d), jnp.bfloat16),
            pltpu.VMEM((N_Y - 1, C, d), jnp.bfloat16),
            pltpu.SemaphoreType.DMA((K,)),
            pltpu.SemaphoreType.DMA((N_Y - 1,)),
            pltpu.SemaphoreType.DMA((N_Y - 1,)),
            pltpu.SemaphoreType.DMA((2 * (N_Y - 1),)),
            pltpu.SemaphoreType.DMA((2 * (N_Y - 1),)),
            pltpu.SemaphoreType.DMA((2 * N_Y,)),
            pltpu.SemaphoreType.DMA((2 * N_Y,)),
        ],
        compiler_params=pltpu.CompilerParams(collective_id=0),
    )(safe, scale, E)


# device time: 50380 ns/iter; 1.1282x vs baseline; 1.1282x over previous
import jax
import jax.numpy as jnp
from jax import lax
from jax.experimental import pallas as pl
from jax.experimental.pallas import tpu as pltpu

N_Y = 4
V_PER = 8192
HALF = 512
C = 128
K = 32


def kernel(ids, E):
    my_y = lax.axis_index("y")
    my_x = lax.axis_index("x")

    my_ids = lax.dynamic_slice(ids, (my_x * HALF,), (HALF,))
    local = my_ids - my_y * V_PER
    in_range = (local >= 0) & (local < V_PER)
    safe = jnp.clip(local, 0, V_PER - 1).astype(jnp.int32)
    scale = in_range.astype(jnp.float32)[:, None]

    t, d = ids.shape[0], E.shape[1]

    def body(ids_ref, scale_ref, e_ref, out_ref,
             fbuf, acc_ref, gbuf, xbuf, rbuf, gsem,
             rs_send, rs_recv, ag_send, ag_recv, x_send, x_recv):
        xx = lax.axis_index("x")
        yy = lax.axis_index("y")
        zz = lax.axis_index("z")
        mine0 = xx * HALF
        theirs0 = (1 - xx) * HALF

        barrier_sem = pltpu.get_barrier_semaphore()
        for j in range(N_Y - 1):
            pl.semaphore_signal(
                barrier_sem, inc=1,
                device_id=(xx, (yy + 1 + j) % N_Y, zz),
                device_id_type=pl.DeviceIdType.MESH,
            )
        pl.semaphore_signal(
            barrier_sem, inc=1,
            device_id=(1 - xx, yy, zz),
            device_id_type=pl.DeviceIdType.MESH,
        )
        pl.semaphore_wait(barrier_sem, N_Y)

        def row_dma(row):
            return pltpu.make_async_copy(
                e_ref.at[pl.ds(ids_ref[row], 1), :],
                fbuf.at[pl.ds(row, 1), :],
                gsem.at[row % K],
            )

        def gather_chunk(c):
            base = c * C

            def gbody(i, _):
                row_dma(base + i).start()

                @pl.when(i >= K)
                def _():
                    row_dma(base + i - K).wait()

                return 0

            lax.fori_loop(0, C, gbody, 0, unroll=8)
            for jj in range(K):
                row_dma(base + C - K + jj).wait()
            acc_ref[pl.ds(base, C), :] = (
                fbuf[pl.ds(base, C), :] * scale_ref[pl.ds(base, C), :]
            ).astype(jnp.bfloat16)

        pending = []
        x_slots = []

        def x_forward(chunk, slot):
            rd = pltpu.make_async_remote_copy(
                src_ref=gbuf.at[pl.ds(chunk * C, C), :],
                dst_ref=xbuf.at[pl.ds(chunk * C, C), :],
                send_sem=x_send.at[slot],
                recv_sem=x_recv.at[slot],
                device_id=(1 - xx, yy, zz),
                device_id_type=pl.DeviceIdType.MESH,
            )
            rd.start()
            x_slots.append((rd, chunk * C))

        for j in range(N_Y - 1):
            c = (yy + 1 + j) % N_Y
            gather_chunk(c)
            rd = pltpu.make_async_remote_copy(
                src_ref=acc_ref.at[pl.ds(c * C, C), :],
                dst_ref=rbuf.at[2 - j],
                send_sem=rs_send.at[j],
                recv_sem=rs_recv.at[2 - j],
                device_id=(xx, c, zz),
                device_id_type=pl.DeviceIdType.MESH,
            )
            rd.start()
            pending.append(rd)
        gather_chunk_own = yy
        gather_chunk(gather_chunk_own)

        for idx in range(N_Y - 1):
            pltpu.make_async_remote_copy(
                src_ref=acc_ref.at[pl.ds(yy * C, C), :],
                dst_ref=rbuf.at[idx],
                send_sem=rs_send.at[idx],
                recv_sem=rs_recv.at[idx],
                device_id=(xx, yy, zz),
                device_id_type=pl.DeviceIdType.MESH,
            ).wait_recv()
        gbuf[pl.ds(yy * C, C), :] = (
            acc_ref[pl.ds(yy * C, C), :]
            + rbuf[0] + rbuf[1] + rbuf[2]
        )

        x_forward(yy, 0)
        out_ref[pl.ds(mine0 + yy * C, C), :] = (
            gbuf[pl.ds(yy * C, C), :].astype(jnp.float32)
        )
        for j in range(N_Y - 1):
            tgt = (yy + 1 + j) % N_Y
            rd = pltpu.make_async_remote_copy(
                src_ref=gbuf.at[pl.ds(yy * C, C), :],
                dst_ref=gbuf.at[pl.ds(yy * C, C), :],
                send_sem=ag_send.at[j],
                recv_sem=ag_recv.at[2 - j],
                device_id=(xx, tgt, zz),
                device_id_type=pl.DeviceIdType.MESH,
            )
            rd.start()
            pending.append(rd)
        for j in range(N_Y - 1):
            c = (yy + 1 + j) % N_Y
            pltpu.make_async_remote_copy(
                src_ref=gbuf.at[pl.ds(c * C, C), :],
                dst_ref=gbuf.at[pl.ds(c * C, C), :],
                send_sem=ag_send.at[j],
                recv_sem=ag_recv.at[j],
                device_id=(xx, yy, zz),
                device_id_type=pl.DeviceIdType.MESH,
            ).wait_recv()
            x_forward(c, j + 1)
            out_ref[pl.ds(mine0 + c * C, C), :] = (
                gbuf[pl.ds(c * C, C), :].astype(jnp.float32)
            )

        for rd, row in x_slots:
            rd.wait_recv()
            out_ref[pl.ds(theirs0 + row, C), :] = (
                xbuf[pl.ds(row, C), :].astype(jnp.float32)
            )
        for rd, _ in x_slots:
            rd.wait_send()
        for rd in pending:
            rd.wait_send()

    return pl.pallas_call(
        body,
        out_shape=jax.ShapeDtypeStruct((t, d), jnp.float32),
        in_specs=[
            pl.BlockSpec(memory_space=pltpu.SMEM),
            pl.BlockSpec(memory_space=pltpu.VMEM),
            pl.BlockSpec(memory_space=pl.ANY),
        ],
        out_specs=pl.BlockSpec(memory_space=pltpu.VMEM),
        scratch_shapes=[
            pltpu.VMEM((HALF, d), jnp.float32),
            pltpu.VMEM((HALF, d), jnp.bfloat16),
            pltpu.VMEM((HALF, d), jnp.bfloat16),
            pltpu.VMEM((HALF, d), jnp.bfloat16),
            pltpu.VMEM((N_Y - 1, C, d), jnp.bfloat16),
            pltpu.SemaphoreType.DMA((K,)),
            pltpu.SemaphoreType.DMA((N_Y - 1,)),
            pltpu.SemaphoreType.DMA((N_Y - 1,)),
            pltpu.SemaphoreType.DMA((N_Y - 1,)),
            pltpu.SemaphoreType.DMA((N_Y - 1,)),
            pltpu.SemaphoreType.DMA((N_Y,)),
            pltpu.SemaphoreType.DMA((N_Y,)),
        ],
        compiler_params=pltpu.CompilerParams(collective_id=0),
    )(safe, scale, E)
